# baseline (speedup 1.0000x reference)
import jax
import jax.numpy as jnp
from jax import lax
from jax.experimental import pallas as pl
from jax.experimental.pallas import tpu as pltpu

_G = 32
_B = 128
_NCHUNK = 2


def _stem_body(p_ref, w_ref, s_ref, t_ref, o_ref):
    acc = jnp.dot(p_ref[...], w_ref[...], preferred_element_type=jnp.float32)
    o_ref[...] = jnp.maximum(acc * s_ref[...] + t_ref[...], 0.0)


def _stem(patches, w, scale, shift):
    m, k = patches.shape
    _, n = w.shape
    mc = m // _NCHUNK
    return pl.pallas_call(
        _stem_body,
        out_shape=jax.ShapeDtypeStruct((m, n), jnp.float32),
        grid=(_NCHUNK,),
        in_specs=[pl.BlockSpec((mc, k), lambda i: (i, 0)),
                  pl.BlockSpec((k, n), lambda i: (0, 0)),
                  pl.BlockSpec((1, n), lambda i: (0, 0)),
                  pl.BlockSpec((1, n), lambda i: (0, 0))],
        out_specs=pl.BlockSpec((mc, n), lambda i: (i, 0)),
        compiler_params=pltpu.CompilerParams(
            dimension_semantics=("parallel",)),
    )(patches, w, scale, shift)


def _conv3x3_patches(bott, m, h, w):
    need_mask = (h > 1) or (w > 1)
    if need_mask:
        r = lax.broadcasted_iota(jnp.int32, (m, 1), 0)
        xq = r % w
        yq = (r // w) % h
    cols = []
    for oy in (-1, 0, 1):
        for ox in (-1, 0, 1):
            if abs(ox) >= w or abs(oy) >= h:
                cols.append(None)
                continue
            d = oy * w + ox
            shifted = bott if d == 0 else pltpu.roll(bott, (-d) % m, 0)
            conds = []
            if ox > 0:
                conds.append(xq < w - ox)
            if ox < 0:
                conds.append(xq >= -ox)
            if oy > 0:
                conds.append(yq < h - oy)
            if oy < 0:
                conds.append(yq >= -oy)
            if conds:
                valid = conds[0]
                for extra in conds[1:]:
                    valid = valid & extra
                shifted = jnp.where(valid, shifted, 0.0)
            cols.append(shifted.astype(jnp.bfloat16))
    return cols


def _make_block_body(n_layers, c_in, nc, h, w, is_head):
    m = nc * h * w
    c_total = c_in + n_layers * _G

    def body(xin_ref, w1_ref, s1_ref, t1_ref, w2_ref, s2_ref, t2_ref,
             ts_ref, tt_ref, tw_ref, tb_ref, out_ref, slab_ref):
        l = pl.program_id(1)

        @pl.when(l == 0)
        def _init():
            slab_ref[...] = jnp.zeros_like(slab_ref)

        pieces = [xin_ref[...]] + [slab_ref[j] for j in range(n_layers)]
        araw = jnp.concatenate(pieces, axis=-1)

        a = jnp.maximum(araw * s1_ref[0] + t1_ref[0], 0.0)
        bott = jnp.dot(a.astype(jnp.bfloat16), w1_ref[0],
                       preferred_element_type=jnp.float32)
        bott = jnp.maximum(bott * s2_ref[0] + t2_ref[0], 0.0)

        if h == 1 and w == 1:
            new = jnp.dot(bott.astype(jnp.bfloat16),
                          w2_ref[0][4 * _B:5 * _B, :],
                          preferred_element_type=jnp.float32)
        else:
            cols = _conv3x3_patches(bott, m, h, w)
            cols = [c if c is not None else jnp.zeros((m, _B), jnp.bfloat16)
                    for c in cols]
            patches = jnp.concatenate(cols, axis=-1)
            new = jnp.dot(patches, w2_ref[0],
                          preferred_element_type=jnp.float32)

        slab_ref[l] = new

        @pl.when(l == n_layers - 1)
        def _finish():
            full = jnp.concatenate([araw[:, :c_total - _G], new], axis=-1)
            act = jnp.maximum(full * ts_ref[...] + tt_ref[...], 0.0)
            y = jnp.dot(act.astype(jnp.bfloat16), tw_ref[...],
                        preferred_element_type=jnp.float32)
            if is_head:
                out_ref[...] = y + tb_ref[...]
            else:
                c_out = y.shape[-1]
                y = y.reshape(m // 2, 2, c_out).mean(axis=1)
                y = y.reshape(nc * h // 2, 2, (w // 2) * c_out).mean(axis=1)
                out_ref[...] = y.reshape(m // 4, c_out)

    return body


def _dense_block_fused(x2d, bp, tail, nc_imgs, h, w, is_head):
    m, c_in = x2d.shape
    n_layers = bp["w1"].shape[0]
    c_total = c_in + n_layers * _G
    mc = m // _NCHUNK
    nc = nc_imgs // _NCHUNK
    if is_head:
        c_out = tail["w"].shape[1]
        out_rows, out_rows_c = nc_imgs, nc
    else:
        c_out = tail["w"].shape[1]
        out_rows, out_rows_c = m // 4, mc // 4
    body = _make_block_body(n_layers, c_in, nc, h, w, is_head)
    return pl.pallas_call(
        body,
        out_shape=jax.ShapeDtypeStruct((out_rows, c_out), jnp.float32),
        grid=(_NCHUNK, n_layers),
        in_specs=[
            pl.BlockSpec((mc, c_in), lambda i, l: (i, 0)),
            pl.BlockSpec((1, c_total, _B), lambda i, l: (l, 0, 0)),
            pl.BlockSpec((1, 1, c_total), lambda i, l: (l, 0, 0)),
            pl.BlockSpec((1, 1, c_total), lambda i, l: (l, 0, 0)),
            pl.BlockSpec((1, 9 * _B, _G), lambda i, l: (l, 0, 0)),
            pl.BlockSpec((1, 1, _B), lambda i, l: (l, 0, 0)),
            pl.BlockSpec((1, 1, _B), lambda i, l: (l, 0, 0)),
            pl.BlockSpec((1, c_total), lambda i, l: (0, 0)),
            pl.BlockSpec((1, c_total), lambda i, l: (0, 0)),
            pl.BlockSpec((c_total, c_out), lambda i, l: (0, 0)),
            pl.BlockSpec(tail["b"].shape, lambda i, l: (0, 0)),
        ],
        out_specs=pl.BlockSpec((out_rows_c, c_out), lambda i, l: (i, 0)),
        scratch_shapes=[pltpu.VMEM((n_layers, mc, _G), jnp.float32)],
        compiler_params=pltpu.CompilerParams(
            dimension_semantics=("parallel", "arbitrary")),
    )(x2d, bp["w1"], bp["s1"], bp["t1"], bp["w2"], bp["s2"], bp["t2"],
      tail["s"], tail["t"], tail["w"], tail["b"])


def _im2col(x, k, stride, pad):
    n, hh, ww, c = x.shape
    xp = jnp.pad(x, ((0, 0), (pad, pad), (pad, pad), (0, 0)))
    ho = (hh + 2 * pad - k) // stride + 1
    wo = (ww + 2 * pad - k) // stride + 1
    cols = []
    for dy in range(k):
        for dx in range(k):
            cols.append(xp[:, dy:dy + stride * ho:stride,
                           dx:dx + stride * wo:stride, :])
    return jnp.concatenate(cols, axis=-1).reshape(n * ho * wo, k * k * c), ho, wo


def kernel(x, w0, s0, t0,
           b0_w1, b0_s1, b0_t1, b0_w2, b0_s2, b0_t2,
           b1_w1, b1_s1, b1_t1, b1_w2, b1_s2, b1_t2,
           b2_w1, b2_s1, b2_t1, b2_w2, b2_s2, b2_t2,
           b3_w1, b3_s1, b3_t1, b3_w2, b3_s2, b3_t2,
           t0_s, t0_t, t0_w,
           t1_s, t1_t, t1_w,
           t2_s, t2_t, t2_w,
           h_s5, h_t5, h_fc_w, h_fc_b):
    blocks = [
        {"w1": b0_w1, "s1": b0_s1, "t1": b0_t1, "w2": b0_w2, "s2": b0_s2, "t2": b0_t2},
        {"w1": b1_w1, "s1": b1_s1, "t1": b1_t1, "w2": b1_w2, "s2": b1_s2, "t2": b1_t2},
        {"w1": b2_w1, "s1": b2_s1, "t1": b2_t1, "w2": b2_w2, "s2": b2_s2, "t2": b2_t2},
        {"w1": b3_w1, "s1": b3_s1, "t1": b3_t1, "w2": b3_w2, "s2": b3_s2, "t2": b3_t2},
    ]
    tails = [
        {"s": t0_s, "t": t0_t, "w": t0_w, "b": jnp.zeros((1, 1), jnp.float32)},
        {"s": t1_s, "t": t1_t, "w": t1_w, "b": jnp.zeros((1, 1), jnp.float32)},
        {"s": t2_s, "t": t2_t, "w": t2_w, "b": jnp.zeros((1, 1), jnp.float32)},
        {"s": h_s5, "t": h_t5, "w": h_fc_w, "b": h_fc_b},
    ]

    xh = jnp.transpose(x, (0, 2, 3, 1)).astype(jnp.float32)
    n_img = xh.shape[0]

    patches, ho, wo = _im2col(xh, 7, 2, 3)
    stem = _stem(patches.astype(jnp.bfloat16), w0, s0, t0)
    xh = stem.reshape(n_img, ho, wo, w0.shape[1])

    xh = lax.reduce_window(xh, -jnp.inf, lax.max, (1, 3, 3, 1), (1, 2, 2, 1),
                           ((0, 0), (1, 1), (1, 1), (0, 0)))

    n, h, w, c = xh.shape
    x2d = xh.reshape(n * h * w, c)
    for bi in range(4):
        is_head = bi == 3
        out = _dense_block_fused(x2d, blocks[bi], tails[bi], n, h, w, is_head)
        if is_head:
            return out
        h, w = h // 2, w // 2
        x2d = out


# speedup vs baseline: 1.2217x; 1.2217x over previous
import jax
import jax.numpy as jnp
from jax import lax
from jax.experimental import pallas as pl
from jax.experimental.pallas import tpu as pltpu

_G = 32
_B = 128
_NCHUNK = 2


def _stem_body(p_ref, w_ref, s_ref, t_ref, o_ref):
    acc = jnp.dot(p_ref[...], w_ref[...], preferred_element_type=jnp.float32)
    o_ref[...] = jnp.maximum(acc * s_ref[...] + t_ref[...], 0.0)


def _stem(patches, w, scale, shift):
    m, k = patches.shape
    _, n = w.shape
    mc = m // _NCHUNK
    return pl.pallas_call(
        _stem_body,
        out_shape=jax.ShapeDtypeStruct((m, n), jnp.float32),
        grid=(_NCHUNK,),
        in_specs=[pl.BlockSpec((mc, k), lambda i: (i, 0)),
                  pl.BlockSpec((k, n), lambda i: (0, 0)),
                  pl.BlockSpec((1, n), lambda i: (0, 0)),
                  pl.BlockSpec((1, n), lambda i: (0, 0))],
        out_specs=pl.BlockSpec((mc, n), lambda i: (i, 0)),
        compiler_params=pltpu.CompilerParams(
            dimension_semantics=("parallel",)),
    )(patches, w, scale, shift)


def _conv3x3_patches(bott, m, h, w):
    need_mask = (h > 1) or (w > 1)
    if need_mask:
        r = lax.broadcasted_iota(jnp.int32, (m, 1), 0)
        xq = r % w
        yq = (r // w) % h
    cols = []
    for oy in (-1, 0, 1):
        for ox in (-1, 0, 1):
            if abs(ox) >= w or abs(oy) >= h:
                cols.append(None)
                continue
            d = oy * w + ox
            shifted = bott if d == 0 else pltpu.roll(bott, (-d) % m, 0)
            conds = []
            if ox > 0:
                conds.append(xq < w - ox)
            if ox < 0:
                conds.append(xq >= -ox)
            if oy > 0:
                conds.append(yq < h - oy)
            if oy < 0:
                conds.append(yq >= -oy)
            if conds:
                valid = conds[0]
                for extra in conds[1:]:
                    valid = valid & extra
                shifted = jnp.where(valid, shifted, 0.0)
            cols.append(shifted.astype(jnp.bfloat16))
    return cols


def _make_block_body(n_layers, c_in, nc, h, w, is_head):
    m = nc * h * w
    c_total = c_in + n_layers * _G

    def body(xin_ref, w1_ref, s1_ref, t1_ref, w2_ref, s2_ref, t2_ref,
             ts_ref, tt_ref, tw_ref, tb_ref, out_ref, slab_ref):
        l = pl.program_id(1)

        @pl.when(l == 0)
        def _init():
            slab_ref[...] = jnp.zeros_like(slab_ref)

        pieces = [xin_ref[...]] + [slab_ref[j] for j in range(n_layers)]
        araw = jnp.concatenate(pieces, axis=-1)

        a = jnp.maximum(araw * s1_ref[0] + t1_ref[0], 0.0)
        bott = jnp.dot(a.astype(jnp.bfloat16), w1_ref[0],
                       preferred_element_type=jnp.float32)
        bott = jnp.maximum(bott * s2_ref[0] + t2_ref[0], 0.0)

        if h == 1 and w == 1:
            new = jnp.dot(bott.astype(jnp.bfloat16),
                          w2_ref[0][4 * _B:5 * _B, :],
                          preferred_element_type=jnp.float32)
        else:
            cols = _conv3x3_patches(bott, m, h, w)
            cols = [c if c is not None else jnp.zeros((m, _B), jnp.bfloat16)
                    for c in cols]
            patches = jnp.concatenate(cols, axis=-1)
            new = jnp.dot(patches, w2_ref[0],
                          preferred_element_type=jnp.float32)

        slab_ref[l] = new

        @pl.when(l == n_layers - 1)
        def _finish():
            full = jnp.concatenate([araw[:, :c_total - _G], new], axis=-1)
            act = jnp.maximum(full * ts_ref[...] + tt_ref[...], 0.0)
            y = jnp.dot(act.astype(jnp.bfloat16), tw_ref[...],
                        preferred_element_type=jnp.float32)
            if is_head:
                out_ref[...] = y + tb_ref[...]
            else:
                c_out = y.shape[-1]
                y = y.reshape(m // 2, 2, c_out).mean(axis=1)
                y = y.reshape(nc * h // 2, 2, (w // 2) * c_out).mean(axis=1)
                out_ref[...] = y.reshape(m // 4, c_out)

    return body


def _dense_block_fused(x2d, bp, tail, nc_imgs, h, w, is_head):
    m, c_in = x2d.shape
    n_layers = bp["w1"].shape[0]
    c_total = c_in + n_layers * _G
    mc = m // _NCHUNK
    nc = nc_imgs // _NCHUNK
    if is_head:
        c_out = tail["w"].shape[1]
        out_rows, out_rows_c = nc_imgs, nc
    else:
        c_out = tail["w"].shape[1]
        out_rows, out_rows_c = m // 4, mc // 4
    body = _make_block_body(n_layers, c_in, nc, h, w, is_head)
    return pl.pallas_call(
        body,
        out_shape=jax.ShapeDtypeStruct((out_rows, c_out), jnp.float32),
        grid=(_NCHUNK, n_layers),
        in_specs=[
            pl.BlockSpec((mc, c_in), lambda i, l: (i, 0)),
            pl.BlockSpec((1, c_total, _B), lambda i, l: (l, 0, 0)),
            pl.BlockSpec((1, 1, c_total), lambda i, l: (l, 0, 0)),
            pl.BlockSpec((1, 1, c_total), lambda i, l: (l, 0, 0)),
            pl.BlockSpec((1, 9 * _B, _G), lambda i, l: (l, 0, 0)),
            pl.BlockSpec((1, 1, _B), lambda i, l: (l, 0, 0)),
            pl.BlockSpec((1, 1, _B), lambda i, l: (l, 0, 0)),
            pl.BlockSpec((1, c_total), lambda i, l: (0, 0)),
            pl.BlockSpec((1, c_total), lambda i, l: (0, 0)),
            pl.BlockSpec((c_total, c_out), lambda i, l: (0, 0)),
            pl.BlockSpec(tail["b"].shape, lambda i, l: (0, 0)),
        ],
        out_specs=pl.BlockSpec((out_rows_c, c_out), lambda i, l: (i, 0)),
        scratch_shapes=[pltpu.VMEM((n_layers, mc, _G), jnp.float32)],
        compiler_params=pltpu.CompilerParams(
            dimension_semantics=("parallel", "arbitrary")),
    )(x2d, bp["w1"], bp["s1"], bp["t1"], bp["w2"], bp["s2"], bp["t2"],
      tail["s"], tail["t"], tail["w"], tail["b"])


def _im2col(x, k, stride, pad):
    n, hh, ww, c = x.shape
    p = lax.conv_general_dilated_patches(
        x, (k, k), (stride, stride), [(pad, pad), (pad, pad)],
        dimension_numbers=("NHWC", "HWIO", "NHWC"))
    ho, wo = p.shape[1], p.shape[2]
    return p.reshape(n * ho * wo, k * k * c), ho, wo


def kernel(x, w0, s0, t0,
           b0_w1, b0_s1, b0_t1, b0_w2, b0_s2, b0_t2,
           b1_w1, b1_s1, b1_t1, b1_w2, b1_s2, b1_t2,
           b2_w1, b2_s1, b2_t1, b2_w2, b2_s2, b2_t2,
           b3_w1, b3_s1, b3_t1, b3_w2, b3_s2, b3_t2,
           t0_s, t0_t, t0_w,
           t1_s, t1_t, t1_w,
           t2_s, t2_t, t2_w,
           h_s5, h_t5, h_fc_w, h_fc_b):
    blocks = [
        {"w1": b0_w1, "s1": b0_s1, "t1": b0_t1, "w2": b0_w2, "s2": b0_s2, "t2": b0_t2},
        {"w1": b1_w1, "s1": b1_s1, "t1": b1_t1, "w2": b1_w2, "s2": b1_s2, "t2": b1_t2},
        {"w1": b2_w1, "s1": b2_s1, "t1": b2_t1, "w2": b2_w2, "s2": b2_s2, "t2": b2_t2},
        {"w1": b3_w1, "s1": b3_s1, "t1": b3_t1, "w2": b3_w2, "s2": b3_s2, "t2": b3_t2},
    ]
    tails = [
        {"s": t0_s, "t": t0_t, "w": t0_w, "b": jnp.zeros((1, 1), jnp.float32)},
        {"s": t1_s, "t": t1_t, "w": t1_w, "b": jnp.zeros((1, 1), jnp.float32)},
        {"s": t2_s, "t": t2_t, "w": t2_w, "b": jnp.zeros((1, 1), jnp.float32)},
        {"s": h_s5, "t": h_t5, "w": h_fc_w, "b": h_fc_b},
    ]

    xh = jnp.transpose(x, (0, 2, 3, 1)).astype(jnp.float32)
    n_img = xh.shape[0]

    w0r = w0.reshape(7, 7, 3, -1).transpose(2, 0, 1, 3).reshape(147, -1)
    patches, ho, wo = _im2col(xh, 7, 2, 3)
    stem = _stem(patches.astype(jnp.bfloat16), w0r, s0, t0)
    xh = stem.reshape(n_img, ho, wo, w0.shape[1])

    xh = lax.reduce_window(xh, -jnp.inf, lax.max, (1, 3, 3, 1), (1, 2, 2, 1),
                           ((0, 0), (1, 1), (1, 1), (0, 0)))

    n, h, w, c = xh.shape
    x2d = xh.reshape(n * h * w, c)
    for bi in range(4):
        is_head = bi == 3
        out = _dense_block_fused(x2d, blocks[bi], tails[bi], n, h, w, is_head)
        if is_head:
            return out
        h, w = h // 2, w // 2
        x2d = out


# speedup vs baseline: 2.7920x; 2.2853x over previous
import jax
import jax.numpy as jnp
from jax import lax
from jax.experimental import pallas as pl
from jax.experimental.pallas import tpu as pltpu

_G = 32
_B = 128
_NCHUNK = 2


def _make_stem_body(nc):
    rows_in = nc * 38
    m_mid = 16 * nc * 16

    def body(x_ref, w_ref, s_ref, t_ref, o_ref):
        xin = x_ref[...]
        vs = []
        for dy in range(7):
            rolled = pltpu.roll(xin, (-dy) % rows_in, 0) if dy else xin
            v = rolled.reshape(rows_in // 2, 2, 114)[:, 0, :]
            v = v.reshape(nc, 19, 114)[:, :16, :].reshape(nc * 16, 114)
            vs.append(v)
        chunks = []
        for xo in range(16):
            parts = [vs[dy][:, 6 * xo:6 * xo + 21] for dy in range(7)]
            chunks.append(jnp.concatenate(parts, axis=-1))
        patches = jnp.concatenate(chunks, axis=0)
        acc = jnp.dot(patches.astype(jnp.bfloat16), w_ref[...],
                      preferred_element_type=jnp.float32)
        stem = jnp.maximum(acc * s_ref[...] + t_ref[...], 0.0)

        r = lax.broadcasted_iota(jnp.int32, (m_mid, 1), 0)
        yq = r % 16
        xq = r // (nc * 16)
        neg = jnp.float32(-jnp.inf)
        colmax = stem
        up = pltpu.roll(stem, m_mid - 1, 0)
        colmax = jnp.maximum(colmax, jnp.where(yq < 15, up, neg))
        dn = pltpu.roll(stem, 1, 0)
        colmax = jnp.maximum(colmax, jnp.where(yq > 0, dn, neg))
        s_x = nc * 16
        mx = colmax
        right = pltpu.roll(colmax, m_mid - s_x, 0)
        mx = jnp.maximum(mx, jnp.where(xq < 15, right, neg))
        left = pltpu.roll(colmax, s_x, 0)
        mx = jnp.maximum(mx, jnp.where(xq > 0, left, neg))
        p = mx.reshape(16 * nc * 8, 2, 64)[:, 0, :]
        p = p.reshape(8, 2, nc * 8, 64)[:, 0]
        p = p.transpose(1, 0, 2).reshape(nc * 64, 64)
        o_ref[...] = p

    return body


def _stem_maxpool(x_nchw, w0, s0, t0):
    n = x_nchw.shape[0]
    nc = n // _NCHUNK
    xh = jnp.transpose(x_nchw, (0, 2, 3, 1)).astype(jnp.float32)
    xp = jnp.pad(xh, ((0, 0), (3, 3), (3, 3), (0, 0)))
    xf = xp.reshape(n * 38, 114)
    return pl.pallas_call(
        _make_stem_body(nc),
        out_shape=jax.ShapeDtypeStruct((n * 64, 64), jnp.float32),
        grid=(_NCHUNK,),
        in_specs=[pl.BlockSpec((nc * 38, 114), lambda i: (i, 0)),
                  pl.BlockSpec((147, 64), lambda i: (0, 0)),
                  pl.BlockSpec((1, 64), lambda i: (0, 0)),
                  pl.BlockSpec((1, 64), lambda i: (0, 0))],
        out_specs=pl.BlockSpec((nc * 64, 64), lambda i: (i, 0)),
        compiler_params=pltpu.CompilerParams(
            dimension_semantics=("parallel",)),
    )(xf, w0, s0, t0)


def _conv3x3_from_taps(z, m, h, w):
    need_mask = (h > 1) or (w > 1)
    if need_mask:
        r = lax.broadcasted_iota(jnp.int32, (m, 1), 0)
        xq = r % w
        yq = (r // w) % h
    acc = None
    t = 0
    for oy in (-1, 0, 1):
        for ox in (-1, 0, 1):
            zt = z[:, t * _G:(t + 1) * _G]
            t += 1
            if abs(ox) >= w or abs(oy) >= h:
                continue
            d = oy * w + ox
            shifted = zt if d == 0 else pltpu.roll(zt, (-d) % m, 0)
            conds = []
            if ox > 0:
                conds.append(xq < w - ox)
            if ox < 0:
                conds.append(xq >= -ox)
            if oy > 0:
                conds.append(yq < h - oy)
            if oy < 0:
                conds.append(yq >= -oy)
            if conds:
                valid = conds[0]
                for extra in conds[1:]:
                    valid = valid & extra
                shifted = jnp.where(valid, shifted, 0.0)
            acc = shifted if acc is None else acc + shifted
    return acc


def _make_block_body(n_layers, c_in, nc, h, w, is_head):
    m = nc * h * w
    c_total = c_in + n_layers * _G

    def body(xin_ref, w1_ref, s1_ref, t1_ref, w2_ref, s2_ref, t2_ref,
             ts_ref, tt_ref, tw_ref, tb_ref, out_ref, araw_ref):
        araw_ref[:, :c_in] = xin_ref[...]
        for l in range(n_layers):
            c0 = c_in + l * _G
            a = jnp.maximum(araw_ref[:, :c0] * s1_ref[l, :, :c0]
                            + t1_ref[l, :, :c0], 0.0)
            bott = jnp.dot(a.astype(jnp.bfloat16), w1_ref[l, :c0, :],
                           preferred_element_type=jnp.float32)
            bott = jnp.maximum(bott * s2_ref[l] + t2_ref[l], 0.0)
            if h == 1 and w == 1:
                new = jnp.dot(bott.astype(jnp.bfloat16),
                              w2_ref[l][:, 4 * _G:5 * _G],
                              preferred_element_type=jnp.float32)
            else:
                z = jnp.dot(bott.astype(jnp.bfloat16), w2_ref[l],
                            preferred_element_type=jnp.float32)
                new = _conv3x3_from_taps(z, m, h, w)
            araw_ref[:, c0:c0 + _G] = new

        act = jnp.maximum(araw_ref[...] * ts_ref[...] + tt_ref[...], 0.0)
        y = jnp.dot(act.astype(jnp.bfloat16), tw_ref[...],
                    preferred_element_type=jnp.float32)
        if is_head:
            out_ref[...] = y + tb_ref[...]
        else:
            c_out = y.shape[-1]
            y = y.reshape(m // 2, 2, c_out).mean(axis=1)
            y = y.reshape(nc * h // 2, 2, (w // 2) * c_out).mean(axis=1)
            out_ref[...] = y.reshape(m // 4, c_out)

    return body


def _dense_block_fused(x2d, bp, tail, nc_imgs, h, w, is_head):
    m, c_in = x2d.shape
    n_layers = bp["w1"].shape[0]
    c_total = c_in + n_layers * _G
    mc = m // _NCHUNK
    nc = nc_imgs // _NCHUNK
    c_out = tail["w"].shape[1]
    if is_head:
        out_rows, out_rows_c = nc_imgs, nc
    else:
        out_rows, out_rows_c = m // 4, mc // 4
    body = _make_block_body(n_layers, c_in, nc, h, w, is_head)
    L = n_layers
    return pl.pallas_call(
        body,
        out_shape=jax.ShapeDtypeStruct((out_rows, c_out), jnp.float32),
        grid=(_NCHUNK,),
        in_specs=[
            pl.BlockSpec((mc, c_in), lambda i: (i, 0)),
            pl.BlockSpec((L, c_total, _B), lambda i: (0, 0, 0)),
            pl.BlockSpec((L, 1, c_total), lambda i: (0, 0, 0)),
            pl.BlockSpec((L, 1, c_total), lambda i: (0, 0, 0)),
            pl.BlockSpec((L, _B, 9 * _G), lambda i: (0, 0, 0)),
            pl.BlockSpec((L, 1, _B), lambda i: (0, 0, 0)),
            pl.BlockSpec((L, 1, _B), lambda i: (0, 0, 0)),
            pl.BlockSpec((1, c_total), lambda i: (0, 0)),
            pl.BlockSpec((1, c_total), lambda i: (0, 0)),
            pl.BlockSpec((c_total, c_out), lambda i: (0, 0)),
            pl.BlockSpec(tail["b"].shape, lambda i: (0, 0)),
        ],
        out_specs=pl.BlockSpec((out_rows_c, c_out), lambda i: (i, 0)),
        scratch_shapes=[pltpu.VMEM((mc, c_total), jnp.float32)],
        compiler_params=pltpu.CompilerParams(
            dimension_semantics=("parallel",)),
    )(x2d, bp["w1"], bp["s1"], bp["t1"], bp["w2"], bp["s2"], bp["t2"],
      tail["s"], tail["t"], tail["w"], tail["b"])


def kernel(x, w0, s0, t0,
           b0_w1, b0_s1, b0_t1, b0_w2, b0_s2, b0_t2,
           b1_w1, b1_s1, b1_t1, b1_w2, b1_s2, b1_t2,
           b2_w1, b2_s1, b2_t1, b2_w2, b2_s2, b2_t2,
           b3_w1, b3_s1, b3_t1, b3_w2, b3_s2, b3_t2,
           t0_s, t0_t, t0_w,
           t1_s, t1_t, t1_w,
           t2_s, t2_t, t2_w,
           h_s5, h_t5, h_fc_w, h_fc_b):
    def _prep_w2(w2):
        L = w2.shape[0]
        return w2.reshape(L, 9, _B, _G).transpose(0, 2, 1, 3).reshape(L, _B, 9 * _G)

    blocks = [
        {"w1": b0_w1, "s1": b0_s1, "t1": b0_t1, "w2": _prep_w2(b0_w2), "s2": b0_s2, "t2": b0_t2},
        {"w1": b1_w1, "s1": b1_s1, "t1": b1_t1, "w2": _prep_w2(b1_w2), "s2": b1_s2, "t2": b1_t2},
        {"w1": b2_w1, "s1": b2_s1, "t1": b2_t1, "w2": _prep_w2(b2_w2), "s2": b2_s2, "t2": b2_t2},
        {"w1": b3_w1, "s1": b3_s1, "t1": b3_t1, "w2": _prep_w2(b3_w2), "s2": b3_s2, "t2": b3_t2},
    ]
    tails = [
        {"s": t0_s, "t": t0_t, "w": t0_w, "b": jnp.zeros((1, 1), jnp.float32)},
        {"s": t1_s, "t": t1_t, "w": t1_w, "b": jnp.zeros((1, 1), jnp.float32)},
        {"s": t2_s, "t": t2_t, "w": t2_w, "b": jnp.zeros((1, 1), jnp.float32)},
        {"s": h_s5, "t": h_t5, "w": h_fc_w, "b": h_fc_b},
    ]

    n = x.shape[0]
    x2d = _stem_maxpool(x, w0, s0, t0)
    h = w = 8
    for bi in range(4):
        is_head = bi == 3
        out = _dense_block_fused(x2d, blocks[bi], tails[bi], n, h, w, is_head)
        if is_head:
            return out
        h, w = h // 2, w // 2
        x2d = out


# speedup vs baseline: 2.9031x; 1.0398x over previous
import jax
import jax.numpy as jnp
from jax import lax
from jax.experimental import pallas as pl
from jax.experimental.pallas import tpu as pltpu

_G = 32
_B = 128
_NCHUNK = 2


def _make_stem_body(nc):
    rows_in = nc * 38
    m_mid = 16 * nc * 16

    def body(x_ref, w_ref, s_ref, t_ref, o_ref):
        xin = x_ref[...]
        vs = []
        for dy in range(7):
            rolled = pltpu.roll(xin, (-dy) % rows_in, 0) if dy else xin
            v = rolled.reshape(rows_in // 2, 2, 114)[:, 0, :]
            v = v.reshape(nc, 19, 114)[:, :16, :].reshape(nc * 16, 114)
            vs.append(v)
        chunks = []
        for xo in range(16):
            parts = [vs[dy][:, 6 * xo:6 * xo + 21] for dy in range(7)]
            chunks.append(jnp.concatenate(parts, axis=-1))
        patches = jnp.concatenate(chunks, axis=0)
        acc = jnp.dot(patches.astype(jnp.bfloat16), w_ref[...],
                      preferred_element_type=jnp.float32)
        stem = jnp.maximum(acc * s_ref[...] + t_ref[...], 0.0)

        r = lax.broadcasted_iota(jnp.int32, (m_mid, 1), 0)
        yq = r % 16
        xq = r // (nc * 16)
        neg = jnp.float32(-jnp.inf)
        colmax = stem
        up = pltpu.roll(stem, m_mid - 1, 0)
        colmax = jnp.maximum(colmax, jnp.where(yq < 15, up, neg))
        dn = pltpu.roll(stem, 1, 0)
        colmax = jnp.maximum(colmax, jnp.where(yq > 0, dn, neg))
        s_x = nc * 16
        mx = colmax
        right = pltpu.roll(colmax, m_mid - s_x, 0)
        mx = jnp.maximum(mx, jnp.where(xq < 15, right, neg))
        left = pltpu.roll(colmax, s_x, 0)
        mx = jnp.maximum(mx, jnp.where(xq > 0, left, neg))
        p = mx.reshape(16 * nc * 8, 2, 64)[:, 0, :]
        p = p.reshape(8, 2, nc * 8, 64)[:, 0]
        p = p.transpose(1, 0, 2).reshape(nc * 64, 64)
        o_ref[...] = p

    return body


def _stem_maxpool(x_nchw, w0, s0, t0):
    n = x_nchw.shape[0]
    nc = n // _NCHUNK
    xh = jnp.transpose(x_nchw, (0, 2, 3, 1)).astype(jnp.float32)
    xp = jnp.pad(xh, ((0, 0), (3, 3), (3, 3), (0, 0)))
    xf = xp.reshape(n * 38, 114)
    return pl.pallas_call(
        _make_stem_body(nc),
        out_shape=jax.ShapeDtypeStruct((n * 64, 64), jnp.float32),
        grid=(_NCHUNK,),
        in_specs=[pl.BlockSpec((nc * 38, 114), lambda i: (i, 0)),
                  pl.BlockSpec((147, 64), lambda i: (0, 0)),
                  pl.BlockSpec((1, 64), lambda i: (0, 0)),
                  pl.BlockSpec((1, 64), lambda i: (0, 0))],
        out_specs=pl.BlockSpec((nc * 64, 64), lambda i: (i, 0)),
        compiler_params=pltpu.CompilerParams(
            dimension_semantics=("parallel",)),
    )(xf, w0, s0, t0)


def _conv3x3_mxu(bott16, w2l, m, h, w):
    r = lax.broadcasted_iota(jnp.int32, (m, 1), 0)
    xq = r % w
    yq = (r // w) % h
    if h > 1:
        pieces = []
        for oy in (-1, 0, 1):
            if oy == 0:
                pieces.append(bott16)
                continue
            sh = pltpu.roll(bott16, (-oy * w) % m, 0)
            cond = (yq < h - oy) if oy > 0 else (yq >= -oy)
            pieces.append(jnp.where(cond, sh, jnp.bfloat16(0)))
        b3 = jnp.concatenate(pieces, axis=-1)
        v = jnp.dot(b3, w2l, preferred_element_type=jnp.float32)
    else:
        v = jnp.dot(bott16, w2l[_B:2 * _B, :],
                    preferred_element_type=jnp.float32)
    out = v[:, _G:2 * _G]
    for ox in (-1, 1):
        if abs(ox) >= w:
            continue
        vx = pltpu.roll(v[:, (ox + 1) * _G:(ox + 2) * _G], (-ox) % m, 0)
        cond = (xq < w - ox) if ox > 0 else (xq >= -ox)
        out = out + jnp.where(cond, vx, 0.0)
    return out


def _make_block_body(n_layers, c_in, nc, h, w, is_head):
    m = nc * h * w
    c_total = c_in + n_layers * _G

    def body(xin_ref, w1_ref, s1_ref, t1_ref, w2_ref, s2_ref, t2_ref,
             ts_ref, tt_ref, tw_ref, tb_ref, out_ref, araw_ref):
        araw_ref[:, :c_in] = xin_ref[...]
        for l in range(n_layers):
            c0 = c_in + l * _G
            a = jnp.maximum(araw_ref[:, :c0] * s1_ref[l, :, :c0]
                            + t1_ref[l, :, :c0], 0.0)
            bott = jnp.dot(a.astype(jnp.bfloat16), w1_ref[l, :c0, :],
                           preferred_element_type=jnp.float32)
            bott = jnp.maximum(bott * s2_ref[l] + t2_ref[l], 0.0)
            new = _conv3x3_mxu(bott.astype(jnp.bfloat16), w2_ref[l], m, h, w)
            araw_ref[:, c0:c0 + _G] = new

        act = jnp.maximum(araw_ref[...] * ts_ref[...] + tt_ref[...], 0.0)
        y = jnp.dot(act.astype(jnp.bfloat16), tw_ref[...],
                    preferred_element_type=jnp.float32)
        if is_head:
            out_ref[...] = y + tb_ref[...]
        else:
            c_out = y.shape[-1]
            y = y.reshape(m // 2, 2, c_out).mean(axis=1)
            y = y.reshape(nc * h // 2, 2, (w // 2) * c_out).mean(axis=1)
            out_ref[...] = y.reshape(m // 4, c_out)

    return body


def _dense_block_fused(x2d, bp, tail, nc_imgs, h, w, is_head):
    m, c_in = x2d.shape
    n_layers = bp["w1"].shape[0]
    c_total = c_in + n_layers * _G
    mc = m // _NCHUNK
    nc = nc_imgs // _NCHUNK
    c_out = tail["w"].shape[1]
    if is_head:
        out_rows, out_rows_c = nc_imgs, nc
    else:
        out_rows, out_rows_c = m // 4, mc // 4
    body = _make_block_body(n_layers, c_in, nc, h, w, is_head)
    L = n_layers
    return pl.pallas_call(
        body,
        out_shape=jax.ShapeDtypeStruct((out_rows, c_out), jnp.float32),
        grid=(_NCHUNK,),
        in_specs=[
            pl.BlockSpec((mc, c_in), lambda i: (i, 0)),
            pl.BlockSpec((L, c_total, _B), lambda i: (0, 0, 0)),
            pl.BlockSpec((L, 1, c_total), lambda i: (0, 0, 0)),
            pl.BlockSpec((L, 1, c_total), lambda i: (0, 0, 0)),
            pl.BlockSpec((L, 3 * _B, 3 * _G), lambda i: (0, 0, 0)),
            pl.BlockSpec((L, 1, _B), lambda i: (0, 0, 0)),
            pl.BlockSpec((L, 1, _B), lambda i: (0, 0, 0)),
            pl.BlockSpec((1, c_total), lambda i: (0, 0)),
            pl.BlockSpec((1, c_total), lambda i: (0, 0)),
            pl.BlockSpec((c_total, c_out), lambda i: (0, 0)),
            pl.BlockSpec(tail["b"].shape, lambda i: (0, 0)),
        ],
        out_specs=pl.BlockSpec((out_rows_c, c_out), lambda i: (i, 0)),
        scratch_shapes=[pltpu.VMEM((mc, c_total), jnp.float32)],
        compiler_params=pltpu.CompilerParams(
            dimension_semantics=("parallel",)),
    )(x2d, bp["w1"], bp["s1"], bp["t1"], bp["w2"], bp["s2"], bp["t2"],
      tail["s"], tail["t"], tail["w"], tail["b"])


def kernel(x, w0, s0, t0,
           b0_w1, b0_s1, b0_t1, b0_w2, b0_s2, b0_t2,
           b1_w1, b1_s1, b1_t1, b1_w2, b1_s2, b1_t2,
           b2_w1, b2_s1, b2_t1, b2_w2, b2_s2, b2_t2,
           b3_w1, b3_s1, b3_t1, b3_w2, b3_s2, b3_t2,
           t0_s, t0_t, t0_w,
           t1_s, t1_t, t1_w,
           t2_s, t2_t, t2_w,
           h_s5, h_t5, h_fc_w, h_fc_b):
    def _prep_w2(w2):
        L = w2.shape[0]
        return (w2.reshape(L, 3, 3, _B, _G).transpose(0, 1, 3, 2, 4)
                .reshape(L, 3 * _B, 3 * _G))

    blocks = [
        {"w1": b0_w1, "s1": b0_s1, "t1": b0_t1, "w2": _prep_w2(b0_w2), "s2": b0_s2, "t2": b0_t2},
        {"w1": b1_w1, "s1": b1_s1, "t1": b1_t1, "w2": _prep_w2(b1_w2), "s2": b1_s2, "t2": b1_t2},
        {"w1": b2_w1, "s1": b2_s1, "t1": b2_t1, "w2": _prep_w2(b2_w2), "s2": b2_s2, "t2": b2_t2},
        {"w1": b3_w1, "s1": b3_s1, "t1": b3_t1, "w2": _prep_w2(b3_w2), "s2": b3_s2, "t2": b3_t2},
    ]
    tails = [
        {"s": t0_s, "t": t0_t, "w": t0_w, "b": jnp.zeros((1, 1), jnp.float32)},
        {"s": t1_s, "t": t1_t, "w": t1_w, "b": jnp.zeros((1, 1), jnp.float32)},
        {"s": t2_s, "t": t2_t, "w": t2_w, "b": jnp.zeros((1, 1), jnp.float32)},
        {"s": h_s5, "t": h_t5, "w": h_fc_w, "b": h_fc_b},
    ]

    n = x.shape[0]
    x2d = _stem_maxpool(x, w0, s0, t0)
    h = w = 8
    for bi in range(4):
        is_head = bi == 3
        out = _dense_block_fused(x2d, blocks[bi], tails[bi], n, h, w, is_head)
        if is_head:
            return out
        h, w = h // 2, w // 2
        x2d = out
